# baseline (speedup 1.0000x reference)
import jax
import jax.numpy as jnp
from jax import lax
from jax.experimental import pallas as pl
from jax.experimental.pallas import tpu as pltpu
from jax.experimental.pallas import tpu_sc as plsc

VOCAB_LEN = 1000000
SEQ_LEN = 2048
H_DIM = 64
BATCH = 4

_NC = 2
_NS = 16
_NW = _NC * _NS
_B = BATCH * SEQ_LEN
_BPW = _B // _NW
_LANES = 16
_NBUF = 8


def _emb_kernel(
    table_hbm, idx_hbm, pos_hbm, out_hbm, idx_s, idx_v, ring, col_v, pos_v, sems
):
    wid = lax.axis_index("s") * _NC + lax.axis_index("c")
    base = wid * _BPW
    pos_base = lax.rem(base, SEQ_LEN)

    pltpu.sync_copy(idx_hbm.at[pl.ds(base, _BPW)], idx_v)

    iota = lax.iota(jnp.int32, _LANES)
    neg = jnp.int32(-2147483648)

    def fill(k, carry):
        v = idx_v[pl.ds(pl.multiple_of(k * _LANES, _LANES), _LANES)]
        for lane in range(_LANES):
            idx_s[k * _LANES + lane] = jnp.max(jnp.where(iota == lane, v, neg))
        return carry

    lax.fori_loop(0, _BPW // _LANES, fill, 0)
    for tb in range(2):
        pltpu.sync_copy(
            pos_hbm.at[:, pl.ds(pos_base + 128 * tb, 128)], pos_v.at[tb]
        )

    def issue(j, slot):
        rt = idx_s[j] >> 7
        off = pl.multiple_of(rt * 128, 128)
        pltpu.async_copy(
            table_hbm.at[:, :, pl.ds(off, 128)], ring.at[slot], sems.at[slot]
        )

    def drain(slot):
        pltpu.make_async_copy(
            table_hbm.at[:, :, pl.ds(0, 128)], ring.at[slot], sems.at[slot]
        ).wait()

    def process(j, slot):
        ri = jnp.full((_LANES,), idx_s[j] & 127, jnp.int32)
        tb = jnp.full((_LANES,), j >> 7, jnp.int32)
        jm = jnp.full((_LANES,), j & 127, jnp.int32)
        for g in range(H_DIM // _LANES):
            c_vec = g * _LANES + iota
            a_vec = c_vec >> 3
            b_vec = c_vec & 7
            tv = plsc.load_gather(ring.at[slot], [a_vec, b_vec, ri])
            pv = plsc.load_gather(pos_v, [tb, c_vec, jm])
            plsc.store_scatter(col_v, [tb, c_vec, jm], tv + pv)

    for s in range(_NBUF):
        issue(jnp.int32(s), s)

    def body(g, carry):
        for s in range(_NBUF):
            j = g * _NBUF + s
            drain(s)
            process(j, s)
            jn = j + _NBUF

            @pl.when(jn < _BPW)
            def _():
                issue(jn, s)

        return carry

    lax.fori_loop(0, _BPW // _NBUF, body, 0)

    b = base // SEQ_LEN
    for tb in range(2):
        pltpu.sync_copy(
            col_v.at[tb], out_hbm.at[b, :, pl.ds(pos_base + 128 * tb, 128)]
        )


@jax.jit
def kernel(x_input, vocab_emb_weight, pos_emb_weight):
    idx_flat = x_input.reshape(-1).astype(jnp.int32)
    table_3d = vocab_emb_weight.T.reshape(H_DIM // 8, 8, VOCAB_LEN)
    mesh = plsc.VectorSubcoreMesh(core_axis_name="c", subcore_axis_name="s")
    run = pl.kernel(
        _emb_kernel,
        out_type=jax.ShapeDtypeStruct((BATCH, H_DIM, SEQ_LEN), jnp.float32),
        mesh=mesh,
        scratch_types=[
            pltpu.SMEM((_BPW,), jnp.int32),
            pltpu.VMEM((_BPW,), jnp.int32),
            pltpu.VMEM((_NBUF, H_DIM // 8, 8, 128), jnp.float32),
            pltpu.VMEM((2, H_DIM, 128), jnp.float32),
            pltpu.VMEM((2, H_DIM, 128), jnp.float32),
            pltpu.SemaphoreType.DMA((_NBUF,)),
        ],
        compiler_params=pltpu.CompilerParams(
            needs_layout_passes=False,
            disable_bounds_checks=True,
        ),
    )
    out_t = run(table_3d, idx_flat, pos_emb_weight.T)
    return out_t.transpose(0, 2, 1)
